# baseline (speedup 1.0000x reference)
import functools
import math

import jax
import jax.numpy as jnp
from jax.experimental import pallas as pl
from jax.experimental.pallas import tpu as pltpu

_B, _H, _N, _CH = 8, 12, 577, 64
_RC, _RN = 32, 72
_BASIS_THRESHOLD = 0.02
_COEF_TOPK = 8
_ATTN_BUDGET = math.ceil(0.25 * _N)
_SCALE = _H ** (-0.5)


def _body(q_ref, k_ref, wq_ref, bq_ref, wk_ref, bk_ref, pn_ref, pbn_ref,
          coef_ref, approx_ref, mask_ref):
    f32 = jnp.float32
    qm = q_ref[0, 0]
    km = k_ref[0, 0]
    wq = wq_ref[...]
    wk = wk_ref[...]
    bq = bq_ref[...]
    bk = bk_ref[...]
    pn = pn_ref[...]
    pbn = pbn_ref[...]

    dn = (((1,), (1,)), ((), ()))
    qp = jax.lax.dot_general(qm, wq, dn, preferred_element_type=f32) + bq
    kw = jax.lax.dot_general(km, wk, dn, preferred_element_type=f32) + bk
    kp = jax.lax.dot_general(kw, pn, (((0,), (0,)), ((), ())),
                             preferred_element_type=f32)
    cheap = jax.lax.dot_general(qp, kp, (((1,), (0,)), ((), ())),
                                preferred_element_type=f32) * _SCALE
    cheap = cheap[1:]

    mx = jnp.max(cheap, axis=-1, keepdims=True)
    ex = jnp.exp(cheap - mx)
    coef = ex / jnp.sum(ex, axis=-1, keepdims=True)

    t = jnp.full((_N - 1, 1), jnp.inf, f32)
    for _ in range(_COEF_TOPK):
        t = jnp.max(jnp.where(coef < t, coef, -jnp.inf), axis=-1, keepdims=True)
    coef_s = jnp.where(coef >= t, coef, 0.0)
    coef_ref[0, 0] = coef_s

    ab = jnp.abs(pbn)
    basis = jnp.where(ab > _BASIS_THRESHOLD, ab, 0.0)
    approx = jax.lax.dot_general(coef_s, basis, (((1,), (1,)), ((), ())),
                                 preferred_element_type=f32)
    approx_ref[0, 0] = approx

    bits = jax.lax.bitcast_convert_type(approx, jnp.int32)
    h16 = jax.lax.shift_right_logical(bits, 15).astype(jnp.int16)
    l16 = (bits & 0x7FFF).astype(jnp.int16)
    ones_col = jnp.ones((_N, 1), jnp.bfloat16)
    one_b = jnp.bfloat16(1.0)
    zero_b = jnp.bfloat16(0.0)

    def count_ge(data, t16):
        sel = jnp.where(data >= t16, one_b, zero_b)
        return jax.lax.dot_general(sel, ones_col, (((1,), (0,)), ((), ())),
                                   preferred_element_type=f32)

    def mk_step(data, budget):
        def step(_, carry):
            lo, hi, c_hi = carry
            mid = lo + jax.lax.shift_right_logical(hi - lo + 1, 1)
            cnt = count_ge(data, mid.astype(jnp.int16))
            ok = cnt >= budget
            return (jnp.where(ok, mid, lo), jnp.where(ok, hi, mid - 1),
                    jnp.where(ok, c_hi, cnt))
        return step

    lo0 = jnp.zeros((_N - 1, 1), jnp.int32)
    hi0 = jnp.full((_N - 1, 1), (1 << 15) - 1, jnp.int32)
    c0 = jnp.zeros((_N - 1, 1), f32)
    tA, _, c_hiA = jax.lax.fori_loop(0, 15, mk_step(h16, float(_ATTN_BUDGET)),
                                     (lo0, hi0, c0))
    tA16 = tA.astype(jnp.int16)
    cand = jnp.where(h16 == tA16, l16, jnp.int16(-1))
    k2 = jnp.float32(_ATTN_BUDGET) - c_hiA
    tB, _, _ = jax.lax.fori_loop(0, 15, mk_step(cand, k2), (lo0, hi0, c0))
    thr = jax.lax.shift_left(tA, 15) | tB
    mask = (bits >= thr).astype(f32)

    mask_ref[0, 0, 0, :] = jnp.ones((_N,), f32)
    mask_ref[0, 0, 1:, :] = mask


@jax.jit
def kernel(q, k, Wq, bq, Wk, bk, proj_n, proj_back_n):
    bq2 = bq.reshape(1, _RC)
    bk2 = bk.reshape(1, _RC)
    rep = lambda i, j: (0, 0)
    grid = (_B, _H)
    out = pl.pallas_call(
        _body,
        grid=grid,
        in_specs=[
            pl.BlockSpec((1, 1, _N, _CH), lambda i, j: (i, j, 0, 0)),
            pl.BlockSpec((1, 1, _N, _CH), lambda i, j: (i, j, 0, 0)),
            pl.BlockSpec((_RC, _CH), rep),
            pl.BlockSpec((1, _RC), rep),
            pl.BlockSpec((_RC, _CH), rep),
            pl.BlockSpec((1, _RC), rep),
            pl.BlockSpec((_N, _RN), rep),
            pl.BlockSpec((_N, _RN), rep),
        ],
        out_specs=[
            pl.BlockSpec((1, 1, _N - 1, _RN), lambda i, j: (i, j, 0, 0)),
            pl.BlockSpec((1, 1, _N - 1, _N), lambda i, j: (i, j, 0, 0)),
            pl.BlockSpec((1, 1, _N, _N), lambda i, j: (i, j, 0, 0)),
        ],
        out_shape=[
            jax.ShapeDtypeStruct((_B, _H, _N - 1, _RN), jnp.float32),
            jax.ShapeDtypeStruct((_B, _H, _N - 1, _N), jnp.float32),
            jax.ShapeDtypeStruct((_B, _H, _N, _N), jnp.float32),
        ],
    )(q, k, Wq, bq2, Wk, bk2, proj_n, proj_back_n)
    coef_s, approx, attn_mask = out
    return (coef_s, approx, attn_mask)


# speedup vs baseline: 1.0724x; 1.0724x over previous
import functools
import math

import jax
import jax.numpy as jnp
from jax.experimental import pallas as pl
from jax.experimental.pallas import tpu as pltpu

_B, _H, _N, _CH = 8, 12, 577, 64
_RC, _RN = 32, 72
_BASIS_THRESHOLD = 0.02
_COEF_TOPK = 8
_ATTN_BUDGET = math.ceil(0.25 * _N)
_SCALE = _H ** (-0.5)


def _body(q_ref, k_ref, wq_ref, bq_ref, wk_ref, bk_ref, pn_ref, pbn_ref,
          coef_ref, approx_ref, mask_ref):
    f32 = jnp.float32
    qm = q_ref[0, 0]
    km = k_ref[0, 0]
    wq = wq_ref[...]
    wk = wk_ref[...]
    bq = bq_ref[...]
    bk = bk_ref[...]
    pn = pn_ref[...]
    pbn = pbn_ref[...]

    dn = (((1,), (1,)), ((), ()))
    qp = jax.lax.dot_general(qm, wq, dn, preferred_element_type=f32) + bq
    kw = jax.lax.dot_general(km, wk, dn, preferred_element_type=f32) + bk
    kp = jax.lax.dot_general(kw, pn, (((0,), (0,)), ((), ())),
                             preferred_element_type=f32)
    cheap = jax.lax.dot_general(qp, kp, (((1,), (0,)), ((), ())),
                                preferred_element_type=f32) * _SCALE
    cheap = cheap[1:]

    mx = jnp.max(cheap, axis=-1, keepdims=True)
    ex = jnp.exp(cheap - mx)
    coef = ex / jnp.sum(ex, axis=-1, keepdims=True)

    t = jnp.full((_N - 1, 1), jnp.inf, f32)
    for _ in range(_COEF_TOPK):
        t = jnp.max(jnp.where(coef < t, coef, -jnp.inf), axis=-1, keepdims=True)
    coef_s = jnp.where(coef >= t, coef, 0.0)
    coef_ref[0, 0] = coef_s

    ab = jnp.abs(pbn)
    basis = jnp.where(ab > _BASIS_THRESHOLD, ab, 0.0)
    approx = jax.lax.dot_general(coef_s, basis, (((1,), (1,)), ((), ())),
                                 preferred_element_type=f32)
    approx_ref[0, 0] = approx

    bits = jax.lax.bitcast_convert_type(approx, jnp.int32)
    ones_col = jnp.ones((_N, 1), jnp.bfloat16)

    def step(_, carry):
        lo, hi = carry
        mid = lo + jax.lax.shift_right_logical(hi - lo + 1, 1)
        sel = (bits >= mid).astype(jnp.bfloat16)
        cnt = jax.lax.dot_general(sel, ones_col, (((1,), (0,)), ((), ())),
                                  preferred_element_type=f32)
        ok = cnt >= float(_ATTN_BUDGET)
        return jnp.where(ok, mid, lo), jnp.where(ok, hi, mid - 1)

    lo0 = jnp.zeros((_N - 1, 1), jnp.int32)
    hi0 = jnp.full((_N - 1, 1), (1 << 30) - 1, jnp.int32)
    lo, _ = jax.lax.fori_loop(0, 30, step, (lo0, hi0))
    mask = (bits >= lo).astype(f32)

    mask_ref[0, 0, 0, :] = jnp.ones((_N,), f32)
    mask_ref[0, 0, 1:, :] = mask


@jax.jit
def kernel(q, k, Wq, bq, Wk, bk, proj_n, proj_back_n):
    bq2 = bq.reshape(1, _RC)
    bk2 = bk.reshape(1, _RC)
    rep = lambda i, j: (0, 0)
    grid = (_B, _H)
    out = pl.pallas_call(
        _body,
        grid=grid,
        in_specs=[
            pl.BlockSpec((1, 1, _N, _CH), lambda i, j: (i, j, 0, 0)),
            pl.BlockSpec((1, 1, _N, _CH), lambda i, j: (i, j, 0, 0)),
            pl.BlockSpec((_RC, _CH), rep),
            pl.BlockSpec((1, _RC), rep),
            pl.BlockSpec((_RC, _CH), rep),
            pl.BlockSpec((1, _RC), rep),
            pl.BlockSpec((_N, _RN), rep),
            pl.BlockSpec((_N, _RN), rep),
        ],
        out_specs=[
            pl.BlockSpec((1, 1, _N - 1, _RN), lambda i, j: (i, j, 0, 0)),
            pl.BlockSpec((1, 1, _N - 1, _N), lambda i, j: (i, j, 0, 0)),
            pl.BlockSpec((1, 1, _N, _N), lambda i, j: (i, j, 0, 0)),
        ],
        out_shape=[
            jax.ShapeDtypeStruct((_B, _H, _N - 1, _RN), jnp.float32),
            jax.ShapeDtypeStruct((_B, _H, _N - 1, _N), jnp.float32),
            jax.ShapeDtypeStruct((_B, _H, _N, _N), jnp.float32),
        ],
        compiler_params=pltpu.CompilerParams(
            dimension_semantics=("parallel", "parallel")),
    )(q, k, Wq, bq2, Wk, bk2, proj_n, proj_back_n)
    coef_s, approx, attn_mask = out
    return (coef_s, approx, attn_mask)
